# Initial kernel scaffold; baseline (speedup 1.0000x reference)
import jax
import jax.numpy as jnp
from jax.experimental import pallas as pl


def kernel(h, edge_index, coord_diff, ln_g, ln_b, cw1, cb1, cw2, cb2, ew1, eb1, ew2, eb2, aw, ab, nw1, nb1, nw2, nb2):
    raise NotImplementedError("write your pallas kernel here")


# speedup vs baseline: 3.9382x; 3.9382x over previous
import functools

import jax
import jax.numpy as jnp
from jax import lax
from jax.experimental import pallas as pl
from jax.experimental.pallas import tpu as pltpu
from jax.experimental.pallas import tpu_sc as plsc

N = 10000
E = 320000
D = 128
H = 128
W = 128
NWIN = E // W
STRIPE = 624

@functools.cache
def _mesh():
    return plsc.VectorSubcoreMesh(core_axis_name="core", subcore_axis_name="subcore")


def _node_pre_body(h_ref, g_ref, b_ref, ws_ref, wt_ref, hn_ref, a_ref, bt_ref):
    x = h_ref[...]
    mu = jnp.mean(x, axis=-1, keepdims=True)
    xc = x - mu
    var = jnp.mean(xc * xc, axis=-1, keepdims=True)
    hn = xc * lax.rsqrt(var + 1e-5) * g_ref[...] + b_ref[...]
    hn_ref[...] = hn
    a_ref[...] = jnp.dot(hn, ws_ref[...], preferred_element_type=jnp.float32)
    bt_ref[...] = jnp.dot(hn, wt_ref[...], preferred_element_type=jnp.float32)


def _node_pre(h, ln_g, ln_b, ews, ewt):
    bn = 1000
    grid = N // bn
    return pl.pallas_call(
        _node_pre_body,
        grid=(grid,),
        in_specs=[
            pl.BlockSpec((bn, D), lambda i: (i, 0)),
            pl.BlockSpec((1, D), lambda i: (0, 0)),
            pl.BlockSpec((1, D), lambda i: (0, 0)),
            pl.BlockSpec((D, H), lambda i: (0, 0)),
            pl.BlockSpec((D, H), lambda i: (0, 0)),
        ],
        out_specs=[
            pl.BlockSpec((bn, D), lambda i: (i, 0)),
            pl.BlockSpec((bn, H), lambda i: (i, 0)),
            pl.BlockSpec((bn, H), lambda i: (i, 0)),
        ],
        out_shape=[
            jax.ShapeDtypeStruct((N, D), jnp.float32),
            jax.ShapeDtypeStruct((N, H), jnp.float32),
            jax.ShapeDtypeStruct((N, H), jnp.float32),
        ],
    )(h, ln_g, ln_b, ews, ewt)


def _sc_gather(a_tab, b_tab, row2, col2):
    @functools.partial(
        pl.kernel,
        out_type=[
            jax.ShapeDtypeStruct((E, H), jnp.float32),
            jax.ShapeDtypeStruct((E, H), jnp.float32),
        ],
        mesh=_mesh(),
    )
    def k(a_hbm, b_hbm, ri_hbm, ci_hbm, g1_hbm, g2_hbm):
        def body(ri_vmem, ci_vmem, g1_vmem, g2_vmem):
            pltpu.sync_copy(a_hbm.at[ri_vmem.at[0]], g1_vmem)
            pltpu.sync_copy(b_hbm.at[ci_vmem.at[0]], g2_vmem)

        pltpu.emit_pipeline(
            body,
            grid=(NWIN,),
            in_specs=[
                pl.BlockSpec((1, W), lambda i: (i, 0)),
                pl.BlockSpec((1, W), lambda i: (i, 0)),
            ],
            out_specs=[
                pl.BlockSpec((W, H), lambda i: (i, 0)),
                pl.BlockSpec((W, H), lambda i: (i, 0)),
            ],
            core_axis_name=("core", "subcore"),
            dimension_semantics=(pltpu.PARALLEL,),
        )(ri_hbm, ci_hbm, g1_hbm, g2_hbm)

    return k(a_tab, b_tab, row2, col2)


def _edge_mlp_body(cd_ref, g1_ref, g2_ref, cw1_ref, cb1_ref, cw2_ref, cb2_ref,
                   ewc_ref, eb1_ref, ew2_ref, eb2_ref, awr_ref, ab_ref, out_ref):
    cd = cd_ref[...]
    cf = (cd[:, 0:1] * cw1_ref[0:1, :] + cd[:, 1:2] * cw1_ref[1:2, :]
          + cb1_ref[...])
    cf = jnp.maximum(cf, 0.0)
    cf = jnp.maximum(
        jnp.dot(cf, cw2_ref[...], preferred_element_type=jnp.float32)
        + cb2_ref[...], 0.0)
    m = (g1_ref[...] + g2_ref[...]
         + jnp.dot(cf, ewc_ref[...], preferred_element_type=jnp.float32)
         + eb1_ref[...])
    ef1 = jnp.maximum(m, 0.0)
    ef2 = jnp.maximum(
        jnp.dot(ef1, ew2_ref[...], preferred_element_type=jnp.float32)
        + eb2_ref[...], 0.0)
    logit = jnp.sum(ef2 * awr_ref[...], axis=-1, keepdims=True) + ab_ref[...]
    out_ref[...] = ef2 * jax.nn.sigmoid(logit)


def _edge_mlp(coord_diff, g1, g2, cw1, cb1, cw2, cb2, ewc, eb1, ew2, eb2, awr, ab):
    be = 2000
    grid = E // be
    full = lambda i: (0, 0)
    return pl.pallas_call(
        _edge_mlp_body,
        grid=(grid,),
        in_specs=[
            pl.BlockSpec((be, 2), lambda i: (i, 0)),
            pl.BlockSpec((be, H), lambda i: (i, 0)),
            pl.BlockSpec((be, H), lambda i: (i, 0)),
            pl.BlockSpec((2, H), full),
            pl.BlockSpec((1, H), full),
            pl.BlockSpec((H, H), full),
            pl.BlockSpec((1, H), full),
            pl.BlockSpec((H, H), full),
            pl.BlockSpec((1, H), full),
            pl.BlockSpec((H, H), full),
            pl.BlockSpec((1, H), full),
            pl.BlockSpec((1, H), full),
            pl.BlockSpec((1, 1), full),
        ],
        out_specs=pl.BlockSpec((be, H), lambda i: (i, 0)),
        out_shape=jax.ShapeDtypeStruct((E, H), jnp.float32),
    )(coord_diff, g1, g2, cw1, cb1, cw2, cb2, ewc, eb1, ew2, eb2, awr, ab)


def _sc_scatter(efs, row2):
    @functools.partial(
        pl.kernel,
        out_type=jax.ShapeDtypeStruct((2, N, D), jnp.float32),
        mesh=_mesh(),
        scratch_types=[pltpu.VMEM_SHARED((N, D), jnp.float32),
                       pltpu.VMEM((16, D), jnp.float32)],
    )
    def k(e_hbm, ri_hbm, o_hbm, acc, zb):
        c = lax.axis_index("core")
        s = lax.axis_index("subcore")
        base = s * STRIPE

        @pl.loop(0, 16)
        def _(r):
            @pl.loop(0, D, step=16)
            def _(c0):
                zb[r, pl.ds(c0, 16)] = jnp.zeros((16,), jnp.float32)

        @pl.loop(0, STRIPE, step=16)
        def _(j):
            pltpu.sync_copy(zb, acc.at[pl.ds(base + j, 16)])

        @pl.when(s == 15)
        def _():
            pltpu.sync_copy(zb, acc.at[pl.ds(16 * STRIPE, N - 16 * STRIPE)])

        plsc.subcore_barrier()

        def body(e_vmem, ri_vmem):
            pltpu.sync_copy(e_vmem, acc.at[ri_vmem.at[0]], add=True)

        pltpu.emit_pipeline(
            body,
            grid=(NWIN,),
            in_specs=[
                pl.BlockSpec((W, H), lambda i: (i, 0)),
                pl.BlockSpec((1, W), lambda i: (i, 0)),
            ],
            out_specs=[],
            core_axis_name=("core", "subcore"),
            dimension_semantics=(pltpu.PARALLEL,),
        )(e_hbm, ri_hbm)

        plsc.subcore_barrier()
        pltpu.sync_copy(acc.at[pl.ds(base, STRIPE)],
                        o_hbm.at[c].at[pl.ds(base, STRIPE)])

        @pl.when(s == 15)
        def _():
            tail = 16 * STRIPE
            pltpu.sync_copy(acc.at[pl.ds(tail, N - tail)],
                            o_hbm.at[c].at[pl.ds(tail, N - tail)])

    return k(efs, row2)


def _node_post_body(hn_ref, agg_ref, w1a_ref, w1b_ref, nb1_ref, nw2_ref,
                    nb2_ref, out_ref):
    hn = hn_ref[...]
    agg = agg_ref[0] + agg_ref[1]
    t = jnp.maximum(
        jnp.dot(hn, w1a_ref[...], preferred_element_type=jnp.float32)
        + jnp.dot(agg, w1b_ref[...], preferred_element_type=jnp.float32)
        + nb1_ref[...], 0.0)
    out_ref[...] = (hn + jnp.dot(t, nw2_ref[...], preferred_element_type=jnp.float32)
                    + nb2_ref[...])


def _node_post(hn, agg2, w1a, w1b, nb1, nw2, nb2):
    bn = 1000
    grid = N // bn
    full = lambda i: (0, 0)
    return pl.pallas_call(
        _node_post_body,
        grid=(grid,),
        in_specs=[
            pl.BlockSpec((bn, D), lambda i: (i, 0)),
            pl.BlockSpec((2, bn, H), lambda i: (0, i, 0)),
            pl.BlockSpec((D, H), full),
            pl.BlockSpec((H, H), full),
            pl.BlockSpec((1, H), full),
            pl.BlockSpec((H, D), full),
            pl.BlockSpec((1, D), full),
        ],
        out_specs=pl.BlockSpec((bn, D), lambda i: (i, 0)),
        out_shape=jax.ShapeDtypeStruct((N, D), jnp.float32),
    )(hn, agg2, w1a, w1b, nb1, nw2, nb2)


def kernel(h, edge_index, coord_diff, ln_g, ln_b, cw1, cb1, cw2, cb2,
           ew1, eb1, ew2, eb2, aw, ab, nw1, nb1, nw2, nb2):
    row2 = edge_index[0].astype(jnp.int32).reshape(NWIN, W)
    col2 = edge_index[1].astype(jnp.int32).reshape(NWIN, W)

    hn, a_tab, b_tab = _node_pre(
        h, ln_g.reshape(1, D), ln_b.reshape(1, D), ew1[:D], ew1[D:2 * D])

    g1, g2 = _sc_gather(a_tab, b_tab, row2, col2)

    efs = _edge_mlp(
        coord_diff, g1, g2, cw1, cb1.reshape(1, H), cw2, cb2.reshape(1, H),
        ew1[2 * D:], eb1.reshape(1, H), ew2, eb2.reshape(1, H),
        aw.reshape(1, H), ab.reshape(1, 1))

    agg2 = _sc_scatter(efs, row2)

    return _node_post(hn, agg2, nw1[:D], nw1[D:], nb1.reshape(1, H),
                      nw2, nb2.reshape(1, D))


# speedup vs baseline: 4.6258x; 1.1746x over previous
import functools

import jax
import jax.numpy as jnp
from jax import lax
from jax.experimental import pallas as pl
from jax.experimental.pallas import tpu as pltpu
from jax.experimental.pallas import tpu_sc as plsc

N = 10000
E = 320000
D = 128
H = 128
W = 128
NWIN = E // W
STRIPE = 624

@functools.cache
def _mesh():
    return plsc.VectorSubcoreMesh(core_axis_name="core", subcore_axis_name="subcore")


def _node_pre_body(h_ref, g_ref, b_ref, ws_ref, wt_ref, hn_ref, a_ref, bt_ref):
    x = h_ref[...]
    mu = jnp.mean(x, axis=-1, keepdims=True)
    xc = x - mu
    var = jnp.mean(xc * xc, axis=-1, keepdims=True)
    hn = xc * lax.rsqrt(var + 1e-5) * g_ref[...] + b_ref[...]
    hn_ref[...] = hn
    a_ref[...] = jnp.dot(hn, ws_ref[...], preferred_element_type=jnp.float32)
    bt_ref[...] = jnp.dot(hn, wt_ref[...], preferred_element_type=jnp.float32)


def _node_pre(h, ln_g, ln_b, ews, ewt):
    bn = 1000
    grid = N // bn
    return pl.pallas_call(
        _node_pre_body,
        grid=(grid,),
        in_specs=[
            pl.BlockSpec((bn, D), lambda i: (i, 0)),
            pl.BlockSpec((1, D), lambda i: (0, 0)),
            pl.BlockSpec((1, D), lambda i: (0, 0)),
            pl.BlockSpec((D, H), lambda i: (0, 0)),
            pl.BlockSpec((D, H), lambda i: (0, 0)),
        ],
        out_specs=[
            pl.BlockSpec((bn, D), lambda i: (i, 0)),
            pl.BlockSpec((bn, H), lambda i: (i, 0)),
            pl.BlockSpec((bn, H), lambda i: (i, 0)),
        ],
        out_shape=[
            jax.ShapeDtypeStruct((N, D), jnp.float32),
            jax.ShapeDtypeStruct((N, H), jnp.float32),
            jax.ShapeDtypeStruct((N, H), jnp.float32),
        ],
    )(h, ln_g, ln_b, ews, ewt)


def _sc_gather(a_tab, b_tab, row2, col2, win0, nwin):
    ec = nwin * W

    @functools.partial(
        pl.kernel,
        out_type=[
            jax.ShapeDtypeStruct((ec, H), jnp.float32),
            jax.ShapeDtypeStruct((ec, H), jnp.float32),
        ],
        mesh=_mesh(),
    )
    def k(a_hbm, b_hbm, ri_hbm, ci_hbm, g1_hbm, g2_hbm):
        def body(ri_vmem, ci_vmem, g1_vmem, g2_vmem):
            pltpu.sync_copy(a_hbm.at[ri_vmem.at[0]], g1_vmem)
            pltpu.sync_copy(b_hbm.at[ci_vmem.at[0]], g2_vmem)

        pltpu.emit_pipeline(
            body,
            grid=(nwin,),
            in_specs=[
                pl.BlockSpec((1, W), lambda i: (i + win0, 0)),
                pl.BlockSpec((1, W), lambda i: (i + win0, 0)),
            ],
            out_specs=[
                pl.BlockSpec((W, H), lambda i: (i, 0)),
                pl.BlockSpec((W, H), lambda i: (i, 0)),
            ],
            core_axis_name=("core", "subcore"),
            dimension_semantics=(pltpu.PARALLEL,),
        )(ri_hbm, ci_hbm, g1_hbm, g2_hbm)

    return k(a_tab, b_tab, row2, col2)


def _edge_mlp_body(cd_ref, g1_ref, g2_ref, cw1_ref, cb1_ref, cw2_ref, cb2_ref,
                   ewc_ref, eb1_ref, ew2_ref, eb2_ref, awr_ref, ab_ref, out_ref):
    cf = jnp.maximum(
        jnp.dot(cd_ref[...], cw1_ref[...], preferred_element_type=jnp.float32)
        + cb1_ref[...], 0.0)
    cf = jnp.maximum(
        jnp.dot(cf, cw2_ref[...], preferred_element_type=jnp.float32)
        + cb2_ref[...], 0.0)
    m = (g1_ref[...] + g2_ref[...]
         + jnp.dot(cf, ewc_ref[...], preferred_element_type=jnp.float32)
         + eb1_ref[...])
    ef1 = jnp.maximum(m, 0.0)
    ef2 = jnp.maximum(
        jnp.dot(ef1, ew2_ref[...], preferred_element_type=jnp.float32)
        + eb2_ref[...], 0.0)
    logit = jnp.dot(ef2, awr_ref[...], preferred_element_type=jnp.float32) + ab_ref[...]
    out_ref[...] = ef2 * jax.nn.sigmoid(logit)


def _edge_mlp(coord_diff, g1, g2, cw1, cb1, cw2, cb2, ewc, eb1, ew2, eb2, awr, ab,
              boff, nblocks):
    be = 2000
    full = lambda i: (0, 0)
    return pl.pallas_call(
        _edge_mlp_body,
        grid=(nblocks,),
        in_specs=[
            pl.BlockSpec((be, 2), lambda i: (i + boff, 0)),
            pl.BlockSpec((be, H), lambda i: (i, 0)),
            pl.BlockSpec((be, H), lambda i: (i, 0)),
            pl.BlockSpec((2, H), full),
            pl.BlockSpec((1, H), full),
            pl.BlockSpec((H, H), full),
            pl.BlockSpec((1, H), full),
            pl.BlockSpec((H, H), full),
            pl.BlockSpec((1, H), full),
            pl.BlockSpec((H, H), full),
            pl.BlockSpec((1, H), full),
            pl.BlockSpec((H, H), full),
            pl.BlockSpec((1, 1), full),
        ],
        out_specs=pl.BlockSpec((be, H), lambda i: (i, 0)),
        out_shape=jax.ShapeDtypeStruct((nblocks * be, H), jnp.float32),
    )(coord_diff, g1, g2, cw1, cb1, cw2, cb2, ewc, eb1, ew2, eb2, awr, ab)


def _sc_scatter(efs_chunks, row2, win0s, nwin):
    kc = len(efs_chunks)

    @functools.partial(
        pl.kernel,
        out_type=jax.ShapeDtypeStruct((2, N, D), jnp.float32),
        mesh=_mesh(),
        scratch_types=[pltpu.VMEM_SHARED((N, D), jnp.float32),
                       pltpu.VMEM((16, D), jnp.float32)],
    )
    def k(*refs):
        e_hbms = refs[:kc]
        ri_hbm = refs[kc]
        o_hbm = refs[kc + 1]
        acc, zb = refs[kc + 2], refs[kc + 3]
        c = lax.axis_index("core")
        s = lax.axis_index("subcore")
        base = s * STRIPE

        @pl.loop(0, 16)
        def _(r):
            @pl.loop(0, D, step=16)
            def _(c0):
                zb[r, pl.ds(c0, 16)] = jnp.zeros((16,), jnp.float32)

        @pl.loop(0, STRIPE, step=16)
        def _(j):
            pltpu.sync_copy(zb, acc.at[pl.ds(base + j, 16)])

        @pl.when(s == 15)
        def _():
            pltpu.sync_copy(zb, acc.at[pl.ds(16 * STRIPE, N - 16 * STRIPE)])

        plsc.subcore_barrier()

        def body(*vmems):
            for j in range(kc):
                pltpu.sync_copy(vmems[j], acc.at[vmems[kc + j].at[0]], add=True)

        pltpu.emit_pipeline(
            body,
            grid=(nwin,),
            in_specs=(
                [pl.BlockSpec((W, H), lambda i: (i, 0)) for _ in range(kc)]
                + [pl.BlockSpec((1, W), lambda i, w0=w0: (i + w0, 0))
                   for w0 in win0s]
            ),
            out_specs=[],
            core_axis_name=("core", "subcore"),
            dimension_semantics=(pltpu.PARALLEL,),
        )(*e_hbms, *([ri_hbm] * kc))

        plsc.subcore_barrier()
        pltpu.sync_copy(acc.at[pl.ds(base, STRIPE)],
                        o_hbm.at[c].at[pl.ds(base, STRIPE)])

        @pl.when(s == 15)
        def _():
            tail = 16 * STRIPE
            pltpu.sync_copy(acc.at[pl.ds(tail, N - tail)],
                            o_hbm.at[c].at[pl.ds(tail, N - tail)])

    return k(*efs_chunks, row2)


def _node_post_body(hn_ref, *refs):
    agg_refs = refs[:-6]
    w1a_ref, w1b_ref, nb1_ref, nw2_ref, nb2_ref, out_ref = refs[-6:]
    hn = hn_ref[...]
    agg = sum(r[0] + r[1] for r in agg_refs)
    t = jnp.maximum(
        jnp.dot(hn, w1a_ref[...], preferred_element_type=jnp.float32)
        + jnp.dot(agg, w1b_ref[...], preferred_element_type=jnp.float32)
        + nb1_ref[...], 0.0)
    out_ref[...] = (hn + jnp.dot(t, nw2_ref[...], preferred_element_type=jnp.float32)
                    + nb2_ref[...])


def _node_post(hn, aggs, w1a, w1b, nb1, nw2, nb2):
    bn = 1000
    grid = N // bn
    full = lambda i: (0, 0)
    return pl.pallas_call(
        _node_post_body,
        grid=(grid,),
        in_specs=[
            pl.BlockSpec((bn, D), lambda i: (i, 0)),
            *[pl.BlockSpec((2, bn, H), lambda i: (0, i, 0)) for _ in aggs],
            pl.BlockSpec((D, H), full),
            pl.BlockSpec((H, H), full),
            pl.BlockSpec((1, H), full),
            pl.BlockSpec((H, D), full),
            pl.BlockSpec((1, D), full),
        ],
        out_specs=pl.BlockSpec((bn, D), lambda i: (i, 0)),
        out_shape=jax.ShapeDtypeStruct((N, D), jnp.float32),
    )(hn, *aggs, w1a, w1b, nb1, nw2, nb2)


def kernel(h, edge_index, coord_diff, ln_g, ln_b, cw1, cb1, cw2, cb2,
           ew1, eb1, ew2, eb2, aw, ab, nw1, nb1, nw2, nb2):
    row2 = edge_index[0].astype(jnp.int32).reshape(NWIN, W)
    col2 = edge_index[1].astype(jnp.int32).reshape(NWIN, W)

    hn, a_tab, b_tab = _node_pre(
        h, ln_g.reshape(1, D), ln_b.reshape(1, D), ew1[:D], ew1[D:2 * D])

    kc = 4
    nwc = NWIN // kc
    blocks_per_chunk = nwc * W // 2000
    efs_chunks, win0s = [], []
    for j in range(kc):
        g1, g2 = _sc_gather(a_tab, b_tab, row2, col2, j * nwc, nwc)
        efs_chunks.append(_edge_mlp(
            coord_diff, g1, g2, cw1, cb1.reshape(1, H), cw2, cb2.reshape(1, H),
            ew1[2 * D:], eb1.reshape(1, H), ew2, eb2.reshape(1, H),
            jnp.broadcast_to(aw, (H, H)), ab.reshape(1, 1),
            j * blocks_per_chunk, blocks_per_chunk))
        win0s.append(j * nwc)

    aggs = [_sc_scatter(efs_chunks[j:j + 1], row2, win0s[j:j + 1], nwc)
            for j in range(kc)]

    return _node_post(hn, aggs, nw1[:D], nw1[D:], nb1.reshape(1, H),
                      nw2, nb2.reshape(1, D))


# speedup vs baseline: 4.6284x; 1.0005x over previous
import functools

import jax
import jax.numpy as jnp
from jax import lax
from jax.experimental import pallas as pl
from jax.experimental.pallas import tpu as pltpu
from jax.experimental.pallas import tpu_sc as plsc

N = 10000
E = 320000
D = 128
H = 128
W = 128
NWIN = E // W
STRIPE = 624

@functools.cache
def _mesh():
    return plsc.VectorSubcoreMesh(core_axis_name="core", subcore_axis_name="subcore")


def _node_pre_body(h_ref, g_ref, b_ref, ws_ref, wt_ref, hn_ref, a_ref, bt_ref):
    x = h_ref[...]
    mu = jnp.mean(x, axis=-1, keepdims=True)
    xc = x - mu
    var = jnp.mean(xc * xc, axis=-1, keepdims=True)
    hn = xc * lax.rsqrt(var + 1e-5) * g_ref[...] + b_ref[...]
    hn_ref[...] = hn
    a_ref[...] = jnp.dot(hn, ws_ref[...], preferred_element_type=jnp.float32)
    bt_ref[...] = jnp.dot(hn, wt_ref[...], preferred_element_type=jnp.float32)


def _node_pre(h, ln_g, ln_b, ews, ewt):
    bn = 1000
    grid = N // bn
    return pl.pallas_call(
        _node_pre_body,
        grid=(grid,),
        in_specs=[
            pl.BlockSpec((bn, D), lambda i: (i, 0)),
            pl.BlockSpec((1, D), lambda i: (0, 0)),
            pl.BlockSpec((1, D), lambda i: (0, 0)),
            pl.BlockSpec((D, H), lambda i: (0, 0)),
            pl.BlockSpec((D, H), lambda i: (0, 0)),
        ],
        out_specs=[
            pl.BlockSpec((bn, D), lambda i: (i, 0)),
            pl.BlockSpec((bn, H), lambda i: (i, 0)),
            pl.BlockSpec((bn, H), lambda i: (i, 0)),
        ],
        out_shape=[
            jax.ShapeDtypeStruct((N, D), jnp.float32),
            jax.ShapeDtypeStruct((N, H), jnp.float32),
            jax.ShapeDtypeStruct((N, H), jnp.float32),
        ],
    )(h, ln_g, ln_b, ews, ewt)


def _sc_gather(a_tab, b_tab, row2, col2, win0, nwin):
    ec = nwin * W

    @functools.partial(
        pl.kernel,
        out_type=[
            jax.ShapeDtypeStruct((ec, H), jnp.float32),
            jax.ShapeDtypeStruct((ec, H), jnp.float32),
        ],
        mesh=_mesh(),
        scratch_types=[pltpu.SemaphoreType.DMA, pltpu.SemaphoreType.DMA],
    )
    def k(a_hbm, b_hbm, ri_hbm, ci_hbm, g1_hbm, g2_hbm, sem1, sem2):
        def body(ri_vmem, ci_vmem, g1_vmem, g2_vmem):
            c1 = pltpu.async_copy(a_hbm.at[ri_vmem.at[0]], g1_vmem, sem1)
            c2 = pltpu.async_copy(b_hbm.at[ci_vmem.at[0]], g2_vmem, sem2)
            c1.wait()
            c2.wait()

        pltpu.emit_pipeline(
            body,
            grid=(nwin,),
            in_specs=[
                pl.BlockSpec((1, W), lambda i: (i + win0, 0)),
                pl.BlockSpec((1, W), lambda i: (i + win0, 0)),
            ],
            out_specs=[
                pl.BlockSpec((W, H), lambda i: (i, 0)),
                pl.BlockSpec((W, H), lambda i: (i, 0)),
            ],
            core_axis_name=("core", "subcore"),
            dimension_semantics=(pltpu.PARALLEL,),
        )(ri_hbm, ci_hbm, g1_hbm, g2_hbm)

    return k(a_tab, b_tab, row2, col2)


def _edge_mlp_body(cd_ref, g1_ref, g2_ref, cw1_ref, cb1_ref, cw2_ref, cb2_ref,
                   ewc_ref, eb1_ref, ew2_ref, eb2_ref, awr_ref, ab_ref, out_ref):
    cf = jnp.maximum(
        jnp.dot(cd_ref[...], cw1_ref[...], preferred_element_type=jnp.float32)
        + cb1_ref[...], 0.0)
    cf = jnp.maximum(
        jnp.dot(cf, cw2_ref[...], preferred_element_type=jnp.float32)
        + cb2_ref[...], 0.0)
    m = (g1_ref[...] + g2_ref[...]
         + jnp.dot(cf, ewc_ref[...], preferred_element_type=jnp.float32)
         + eb1_ref[...])
    ef1 = jnp.maximum(m, 0.0)
    ef2 = jnp.maximum(
        jnp.dot(ef1, ew2_ref[...], preferred_element_type=jnp.float32)
        + eb2_ref[...], 0.0)
    logit = jnp.dot(ef2, awr_ref[...], preferred_element_type=jnp.float32) + ab_ref[...]
    out_ref[...] = ef2 * jax.nn.sigmoid(logit)


def _edge_mlp(coord_diff, g1, g2, cw1, cb1, cw2, cb2, ewc, eb1, ew2, eb2, awr, ab,
              boff, nblocks):
    be = 2000
    full = lambda i: (0, 0)
    return pl.pallas_call(
        _edge_mlp_body,
        grid=(nblocks,),
        in_specs=[
            pl.BlockSpec((be, 2), lambda i: (i + boff, 0)),
            pl.BlockSpec((be, H), lambda i: (i, 0)),
            pl.BlockSpec((be, H), lambda i: (i, 0)),
            pl.BlockSpec((2, H), full),
            pl.BlockSpec((1, H), full),
            pl.BlockSpec((H, H), full),
            pl.BlockSpec((1, H), full),
            pl.BlockSpec((H, H), full),
            pl.BlockSpec((1, H), full),
            pl.BlockSpec((H, H), full),
            pl.BlockSpec((1, H), full),
            pl.BlockSpec((H, H), full),
            pl.BlockSpec((1, 1), full),
        ],
        out_specs=pl.BlockSpec((be, H), lambda i: (i, 0)),
        out_shape=jax.ShapeDtypeStruct((nblocks * be, H), jnp.float32),
    )(coord_diff, g1, g2, cw1, cb1, cw2, cb2, ewc, eb1, ew2, eb2, awr, ab)


def _sc_scatter(efs_chunks, row2, win0s, nwin):
    kc = len(efs_chunks)

    @functools.partial(
        pl.kernel,
        out_type=jax.ShapeDtypeStruct((2, N, D), jnp.float32),
        mesh=_mesh(),
        scratch_types=[pltpu.VMEM_SHARED((N, D), jnp.float32),
                       pltpu.VMEM((48, D), jnp.float32),
                       pltpu.SemaphoreType.DMA],
    )
    def k(*refs):
        e_hbms = refs[:kc]
        ri_hbm = refs[kc]
        o_hbm = refs[kc + 1]
        acc, zb, zsem = refs[kc + 2], refs[kc + 3], refs[kc + 4]
        c = lax.axis_index("core")
        s = lax.axis_index("subcore")
        base = s * STRIPE

        @pl.loop(0, 48)
        def _(r):
            @pl.loop(0, D, step=16)
            def _(c0):
                zb[r, pl.ds(c0, 16)] = jnp.zeros((16,), jnp.float32)

        zcps = [pltpu.async_copy(zb, acc.at[pl.ds(base + j * 48, 48)], zsem)
                for j in range(STRIPE // 48)]

        @pl.when(s == 15)
        def _():
            pltpu.sync_copy(zb.at[pl.ds(0, N - 16 * STRIPE)],
                            acc.at[pl.ds(16 * STRIPE, N - 16 * STRIPE)])

        for cp in zcps:
            cp.wait()
        plsc.subcore_barrier()

        def body(*vmems):
            for j in range(kc):
                pltpu.sync_copy(vmems[j], acc.at[vmems[kc + j].at[0]], add=True)

        pltpu.emit_pipeline(
            body,
            grid=(nwin,),
            in_specs=(
                [pl.BlockSpec((W, H), lambda i: (i, 0)) for _ in range(kc)]
                + [pl.BlockSpec((1, W), lambda i, w0=w0: (i + w0, 0))
                   for w0 in win0s]
            ),
            out_specs=[],
            core_axis_name=("core", "subcore"),
            dimension_semantics=(pltpu.PARALLEL,),
        )(*e_hbms, *([ri_hbm] * kc))

        plsc.subcore_barrier()
        pltpu.sync_copy(acc.at[pl.ds(base, STRIPE)],
                        o_hbm.at[c].at[pl.ds(base, STRIPE)])

        @pl.when(s == 15)
        def _():
            tail = 16 * STRIPE
            pltpu.sync_copy(acc.at[pl.ds(tail, N - tail)],
                            o_hbm.at[c].at[pl.ds(tail, N - tail)])

    return k(*efs_chunks, row2)


def _node_post_body(hn_ref, *refs):
    agg_refs = refs[:-6]
    w1a_ref, w1b_ref, nb1_ref, nw2_ref, nb2_ref, out_ref = refs[-6:]
    hn = hn_ref[...]
    agg = sum(r[0] + r[1] for r in agg_refs)
    t = jnp.maximum(
        jnp.dot(hn, w1a_ref[...], preferred_element_type=jnp.float32)
        + jnp.dot(agg, w1b_ref[...], preferred_element_type=jnp.float32)
        + nb1_ref[...], 0.0)
    out_ref[...] = (hn + jnp.dot(t, nw2_ref[...], preferred_element_type=jnp.float32)
                    + nb2_ref[...])


def _node_post(hn, aggs, w1a, w1b, nb1, nw2, nb2):
    bn = 1000
    grid = N // bn
    full = lambda i: (0, 0)
    return pl.pallas_call(
        _node_post_body,
        grid=(grid,),
        in_specs=[
            pl.BlockSpec((bn, D), lambda i: (i, 0)),
            *[pl.BlockSpec((2, bn, H), lambda i: (0, i, 0)) for _ in aggs],
            pl.BlockSpec((D, H), full),
            pl.BlockSpec((H, H), full),
            pl.BlockSpec((1, H), full),
            pl.BlockSpec((H, D), full),
            pl.BlockSpec((1, D), full),
        ],
        out_specs=pl.BlockSpec((bn, D), lambda i: (i, 0)),
        out_shape=jax.ShapeDtypeStruct((N, D), jnp.float32),
    )(hn, *aggs, w1a, w1b, nb1, nw2, nb2)


def kernel(h, edge_index, coord_diff, ln_g, ln_b, cw1, cb1, cw2, cb2,
           ew1, eb1, ew2, eb2, aw, ab, nw1, nb1, nw2, nb2):
    row2 = edge_index[0].astype(jnp.int32).reshape(NWIN, W)
    col2 = edge_index[1].astype(jnp.int32).reshape(NWIN, W)

    hn, a_tab, b_tab = _node_pre(
        h, ln_g.reshape(1, D), ln_b.reshape(1, D), ew1[:D], ew1[D:2 * D])

    kc = 4
    nwc = NWIN // kc
    blocks_per_chunk = nwc * W // 2000
    efs_chunks, win0s = [], []
    for j in range(kc):
        g1, g2 = _sc_gather(a_tab, b_tab, row2, col2, j * nwc, nwc)
        efs_chunks.append(_edge_mlp(
            coord_diff, g1, g2, cw1, cb1.reshape(1, H), cw2, cb2.reshape(1, H),
            ew1[2 * D:], eb1.reshape(1, H), ew2, eb2.reshape(1, H),
            jnp.broadcast_to(aw, (H, H)), ab.reshape(1, 1),
            j * blocks_per_chunk, blocks_per_chunk))
        win0s.append(j * nwc)

    aggs = [_sc_scatter(efs_chunks[j:j + 1], row2, win0s[j:j + 1], nwc)
            for j in range(kc)]

    return _node_post(hn, aggs, nw1[:D], nw1[D:], nb1.reshape(1, H),
                      nw2, nb2.reshape(1, D))
